# baseline (speedup 1.0000x reference)
import functools

import jax
import jax.numpy as jnp
from jax import lax
from jax.experimental import pallas as pl
from jax.experimental.pallas import tpu as pltpu
from jax.experimental.pallas import tpu_sc as plsc

CAP = 1000000
DIM = 64
BATCH = 16384
NUM_CORES = 2
ROWS_PER_WORKER = DIM // NUM_CORES

_mesh = plsc.ScalarSubcoreMesh(axis_name="c", num_cores=NUM_CORES)


@functools.partial(
    pl.kernel,
    mesh=_mesh,
    out_type=(),
    scratch_types=[
        pltpu.VMEM_SHARED((ROWS_PER_WORKER, BATCH), jnp.float32),
        pltpu.SemaphoreType.DMA,
    ],
)
def _push(mem_ref, vals_hbm, buf, sem):
    wid = lax.axis_index("c")
    base = wid * ROWS_PER_WORKER
    src = vals_hbm.at[pl.ds(base, ROWS_PER_WORKER), :]
    dst = mem_ref.at[pl.ds(base, ROWS_PER_WORKER), pl.ds(0, BATCH)]
    pltpu.async_copy(src, buf, sem).wait()
    pltpu.sync_copy(buf, dst)


def kernel(memory, values):
    mem_ref = jax.new_ref(memory.T)
    _push(mem_ref, values.T)
    return mem_ref[...].T


# speedup vs baseline: 1.0092x; 1.0092x over previous
import functools

import jax
import jax.numpy as jnp
from jax import lax
from jax.experimental import pallas as pl
from jax.experimental.pallas import tpu as pltpu
from jax.experimental.pallas import tpu_sc as plsc

CAP = 1000000
DIM = 64
BATCH = 16384
NUM_CORES = 2
NUM_SUBCORES = 16
NUM_WORKERS = NUM_CORES * NUM_SUBCORES
ROWS_PER_WORKER = DIM // NUM_WORKERS

_mesh = plsc.VectorSubcoreMesh(core_axis_name="c", subcore_axis_name="s")


@functools.partial(
    pl.kernel,
    mesh=_mesh,
    out_type=(),
    scratch_types=[
        pltpu.VMEM((ROWS_PER_WORKER, BATCH), jnp.float32),
        pltpu.SemaphoreType.DMA,
    ],
)
def _push(mem_ref, vals_hbm, buf, sem):
    wid = lax.axis_index("s") * NUM_CORES + lax.axis_index("c")
    base = wid * ROWS_PER_WORKER
    src = vals_hbm.at[pl.ds(base, ROWS_PER_WORKER), :]
    dst = mem_ref.at[pl.ds(base, ROWS_PER_WORKER), pl.ds(0, BATCH)]
    pltpu.async_copy(src, buf, sem).wait()
    pltpu.sync_copy(buf, dst)


def kernel(memory, values):
    mem_ref = jax.new_ref(memory.T)
    _push(mem_ref, values.T)
    return mem_ref[...].T
